# baseline (speedup 1.0000x reference)
import jax
import jax.numpy as jnp
from jax import lax
from jax.experimental import pallas as pl
from jax.experimental.pallas import tpu as pltpu

E = 8
TOPK = 2
DM = 1024
DFF = 4096
T = 2048
B = 256
NA = T * TOPK
NBMAX = NA // B + (E - 1)
NP = NBMAX * B
BF = jnp.bfloat16


def _routing_kernel(x_ref, wg_ref, pT_ref, pi_ref, wi_ref, meta_ref):
    logits = jnp.dot(x_ref[...], wg_ref[...], preferred_element_type=jnp.float32)
    iota = lax.broadcasted_iota(jnp.int32, (T, E), 1)
    m1 = jnp.max(logits, axis=1, keepdims=True)
    a1 = jnp.min(jnp.where(logits == m1, iota, E), axis=1, keepdims=True)
    masked = jnp.where(iota == a1, -jnp.inf, logits)
    m2 = jnp.max(masked, axis=1, keepdims=True)
    a2 = jnp.min(jnp.where(masked == m2, iota, E), axis=1, keepdims=True)
    z = jnp.exp(m2 - m1)
    w1 = 1.0 / (1.0 + z)
    w2 = z / (1.0 + z)

    m0 = (iota == a1).astype(jnp.float32)
    m1h = (iota == a2).astype(jnp.float32)
    mc = m0 + m1h

    tri = (lax.broadcasted_iota(jnp.int32, (T, T), 0)
           > lax.broadcasted_iota(jnp.int32, (T, T), 1)).astype(BF)
    s_ex = jnp.dot(tri, mc.astype(BF), preferred_element_type=jnp.float32)

    rank1 = jnp.sum(m0 * s_ex, axis=1, keepdims=True)
    rank2 = jnp.sum(m1h * s_ex, axis=1, keepdims=True)

    g = jnp.sum(mc, axis=0, keepdims=True)
    nb = jnp.floor((g + (B - 1)) * (1.0 / B))
    triu = (lax.broadcasted_iota(jnp.int32, (E, E), 0)
            < lax.broadcasted_iota(jnp.int32, (E, E), 1)).astype(BF)
    bs_row = jnp.dot(nb.astype(BF), triu,
                     preferred_element_type=jnp.float32)

    start1 = jnp.sum(m0 * bs_row, axis=1, keepdims=True)
    start2 = jnp.sum(m1h * bs_row, axis=1, keepdims=True)
    p1 = (B * start1 + rank1).astype(jnp.int32)
    p2 = (B * start2 + rank2).astype(jnp.int32)

    pcat = jnp.concatenate([p1, p2], axis=1)
    wcat = jnp.concatenate([w1, w2], axis=1)
    pi_ref[...] = pcat
    wi_ref[...] = wcat
    pT_ref[...] = pcat.T

    nbv = jnp.sum(nb, axis=1, keepdims=True)
    bs_incl = bs_row + nb
    bs_incl_col = bs_incl.reshape(E, 1)
    b_row = lax.broadcasted_iota(jnp.int32, (1, NBMAX), 1).astype(jnp.float32)
    be = jnp.sum((jnp.broadcast_to(b_row, (E, NBMAX))
                  >= bs_incl_col).astype(jnp.float32), axis=0, keepdims=True)
    e_ids = lax.broadcasted_iota(jnp.int32, (1, E), 1).astype(jnp.float32)
    e_last = jnp.max(jnp.where(g > 0, e_ids, 0.0), axis=1, keepdims=True)
    be = jnp.where(b_row < nbv, be, e_last)
    meta_ref[...] = jnp.concatenate([be, nbv], axis=1).astype(jnp.int32)


def _route(x, wg):
    return pl.pallas_call(
        _routing_kernel,
        out_shape=(
            jax.ShapeDtypeStruct((TOPK, T), jnp.int32),
            jax.ShapeDtypeStruct((T, TOPK), jnp.int32),
            jax.ShapeDtypeStruct((T, TOPK), jnp.float32),
            jax.ShapeDtypeStruct((1, NBMAX + 1), jnp.int32),
        ),
    )(x, wg)


def _up_kernel(meta_ref, x_ref, w1_ref, b1_ref, pT_ref, h_ref):
    b = pl.program_id(0)
    nbv = meta_ref[NBMAX]

    @pl.when(b < nbv)
    def _():
        slot = lax.broadcasted_iota(jnp.int32, (B, T), 0) + b * B
        pmat = ((slot == pT_ref[0, :][None, :]) |
                (slot == pT_ref[1, :][None, :])).astype(BF)
        xg = jnp.dot(pmat, x_ref[...], preferred_element_type=jnp.float32)
        w1b = w1_ref[0].astype(BF)
        h = jnp.dot(xg.astype(BF), w1b, preferred_element_type=jnp.float32)
        h = jax.nn.gelu(h + b1_ref[0, 0, :])
        h_ref[...] = h.astype(BF)


def _down_kernel(meta_ref, h_ref, w2_ref, b2_ref, pi_ref, wi_ref, out_ref):
    b = pl.program_id(0)
    nbv = meta_ref[NBMAX]

    @pl.when(b == 0)
    def _():
        out_ref[...] = jnp.zeros_like(out_ref)

    @pl.when(b < nbv)
    def _():
        w2b = w2_ref[0].astype(BF)
        y = jnp.dot(h_ref[...], w2b, preferred_element_type=jnp.float32)
        y = y + b2_ref[0, 0, :]
        slot = lax.broadcasted_iota(jnp.int32, (T, B), 1) + b * B
        smat = (jnp.where(slot == pi_ref[:, 0:1], wi_ref[:, 0:1], 0.0) +
                jnp.where(slot == pi_ref[:, 1:2], wi_ref[:, 1:2], 0.0)
                ).astype(BF)
        out_ref[...] += jnp.dot(smat, y.astype(BF),
                                preferred_element_type=jnp.float32)


@jax.jit
def _moe(inputs, Wg, W1, b1, W2, b2):
    pT, pi, wi, meta2d = _route(inputs, Wg)
    meta = meta2d.reshape(NBMAX + 1)

    x16 = inputs.astype(BF)
    b1r = b1.reshape(E, 1, DFF)
    b2r = b2.reshape(E, 1, DM)

    h = pl.pallas_call(
        _up_kernel,
        grid_spec=pltpu.PrefetchScalarGridSpec(
            num_scalar_prefetch=1,
            grid=(NBMAX,),
            in_specs=[
                pl.BlockSpec((T, DM), lambda b, m: (0, 0)),
                pl.BlockSpec((1, DM, DFF), lambda b, m: (m[b], 0, 0)),
                pl.BlockSpec((1, 1, DFF), lambda b, m: (m[b], 0, 0)),
                pl.BlockSpec((TOPK, T), lambda b, m: (0, 0)),
            ],
            out_specs=pl.BlockSpec((B, DFF), lambda b, m: (b, 0)),
        ),
        out_shape=jax.ShapeDtypeStruct((NP, DFF), BF),
        compiler_params=pltpu.CompilerParams(
            dimension_semantics=("arbitrary",)),
    )(meta, x16, W1, b1r, pT)

    out = pl.pallas_call(
        _down_kernel,
        grid_spec=pltpu.PrefetchScalarGridSpec(
            num_scalar_prefetch=1,
            grid=(NBMAX,),
            in_specs=[
                pl.BlockSpec((B, DFF), lambda b, m: (b, 0)),
                pl.BlockSpec((1, DFF, DM), lambda b, m: (m[b], 0, 0)),
                pl.BlockSpec((1, 1, DM), lambda b, m: (m[b], 0, 0)),
                pl.BlockSpec((T, TOPK), lambda b, m: (0, 0)),
                pl.BlockSpec((T, TOPK), lambda b, m: (0, 0)),
            ],
            out_specs=pl.BlockSpec((T, DM), lambda b, m: (0, 0)),
        ),
        out_shape=jax.ShapeDtypeStruct((T, DM), jnp.float32),
        compiler_params=pltpu.CompilerParams(
            dimension_semantics=("arbitrary",)),
    )(meta, h, W2, b2r, pi, wi)
    return out


def kernel(inputs, Wg, W1, b1, W2, b2):
    return _moe(inputs, Wg, W1, b1, W2, b2)
